# baseline (speedup 1.0000x reference)
import jax
import jax.numpy as jnp
from jax import lax
from jax.experimental import pallas as pl
from jax.experimental.pallas import tpu as pltpu


def _lrelu(x):
    return jnp.where(x >= 0, x, 0.01 * x)


def _s1a_body(x_ref, w1_ref, w2_ref, out_ref):
    x = x_ref[0]
    w1 = w1_ref[0]
    w2 = w2_ref[0]
    h1 = _lrelu(jnp.dot(w1, x, preferred_element_type=jnp.float32))
    h2 = _lrelu(jnp.dot(w2, h1, preferred_element_type=jnp.float32))
    out_ref[0] = h2


def _s1b_body(flat_ref, w3_ref, out_ref):
    out_ref[...] = jnp.dot(w3_ref[...], flat_ref[...],
                           preferred_element_type=jnp.float32)


def _argmax_rows(y):
    m = jnp.max(y, axis=0, keepdims=True)
    ri = lax.broadcasted_iota(jnp.int32, y.shape, 0)
    return jnp.min(jnp.where(y == m, ri, jnp.int32(2147483647)), axis=0)


def _select(z, oh, n_experts, P):
    return jnp.sum(z.reshape(n_experts, 32, P) * oh[:, None, :], axis=0)


def _mlp_all(xr, e, w1_ref, w2_ref, w3_ref, n_experts):
    ci, P = xr.shape
    oh = (lax.broadcasted_iota(jnp.int32, (n_experts, P), 0)
          == e[None, :]).astype(jnp.float32)
    t1 = jnp.swapaxes(w1_ref[...], 1, 2).reshape(n_experts * 32, ci)
    z1 = _lrelu(jnp.dot(t1, xr, preferred_element_type=jnp.float32))
    y1 = _select(z1, oh, n_experts, P)
    t2 = jnp.swapaxes(w2_ref[...], 1, 2).reshape(n_experts * 32, 32)
    z2 = _lrelu(jnp.dot(t2, y1, preferred_element_type=jnp.float32))
    y2 = _select(z2, oh, n_experts, P)
    t3 = jnp.swapaxes(w3_ref[...], 1, 2).reshape(n_experts * 32, 32)
    z3 = jnp.dot(t3, y2, preferred_element_type=jnp.float32)
    return _select(z3, oh, n_experts, P)


def _s23_body(o_ref, x_ref, a1_ref, a2_ref, a3_ref,
              b1_ref, b2_ref, b3_ref, out_ref, i_s, acc_s):
    c = pl.program_id(1)
    xr = x_ref[0]
    P = xr.shape[1]

    @pl.when(c == 0)
    def _stage2():
        e1 = _argmax_rows(o_ref[...])
        y = _mlp_all(xr, e1, a1_ref, a2_ref, a3_ref, 16)
        i2 = _argmax_rows(y)
        inds12 = e1 * 16 + (i2 - 8)
        i_s[0, :] = jnp.clip(inds12, 0, 255)
        i_s[1, :] = inds12
        acc_s[...] = jnp.zeros(acc_s.shape, jnp.float32)

    el = i_s[0, :] - c * 128
    acc_s[...] += _mlp_all(xr, el, b1_ref, b2_ref, b3_ref, 128)

    @pl.when(c == 1)
    def _final():
        i3 = _argmax_rows(acc_s[...])
        out_ref[0, 0, :] = jnp.clip(i_s[1, :] * 16 + (i3 - 8), 0, 4095)


def kernel(x_in, W1, b1, W2, b2, W3, b3, A1w, A1b, A2w, A2b, A3w, A3b,
           B1w, B1b, B2w, B2b, B3w, B3b):
    bs, ch_in, height, width = x_in.shape
    lat = W1.shape[1]
    c0 = 16

    xrows = jnp.transpose(x_in, (0, 2, 1, 3)).reshape(height, ch_in, width)
    h2 = pl.pallas_call(
        _s1a_body,
        grid=(height,),
        in_specs=[
            pl.BlockSpec((1, ch_in, width), lambda r: (r, 0, 0)),
            pl.BlockSpec((1, lat, ch_in), lambda r: (r, 0, 0)),
            pl.BlockSpec((1, lat, lat), lambda r: (r, 0, 0)),
        ],
        out_specs=pl.BlockSpec((1, lat, width), lambda r: (r, 0, 0)),
        out_shape=jax.ShapeDtypeStruct((height, lat, width), jnp.float32),
    )(xrows, W1, W2)

    flat = h2.reshape(height * lat, width)
    o = pl.pallas_call(
        _s1b_body,
        out_shape=jax.ShapeDtypeStruct((height * c0, width), jnp.float32),
    )(flat, W3)

    out = pl.pallas_call(
        _s23_body,
        grid=(height, 2),
        in_specs=[
            pl.BlockSpec((c0, width), lambda r, c: (r, 0)),
            pl.BlockSpec((1, ch_in, width), lambda r, c: (r, 0, 0)),
            pl.BlockSpec((c0, ch_in, 32), lambda r, c: (r, 0, 0)),
            pl.BlockSpec((c0, 32, 32), lambda r, c: (r, 0, 0)),
            pl.BlockSpec((c0, 32, 32), lambda r, c: (r, 0, 0)),
            pl.BlockSpec((128, ch_in, 32), lambda r, c: (2 * r + c, 0, 0)),
            pl.BlockSpec((128, 32, 32), lambda r, c: (2 * r + c, 0, 0)),
            pl.BlockSpec((128, 32, 32), lambda r, c: (2 * r + c, 0, 0)),
        ],
        out_specs=pl.BlockSpec((1, 1, width), lambda r, c: (r, 0, 0)),
        out_shape=jax.ShapeDtypeStruct((height, 1, width), jnp.int32),
        scratch_shapes=[
            pltpu.VMEM((8, width), jnp.int32),
            pltpu.VMEM((lat, width), jnp.float32),
        ],
    )(o, xrows, A1w, A2w, A3w, B1w, B2w, B3w)

    return out.reshape(bs, 1, height, width)


# speedup vs baseline: 1.2550x; 1.2550x over previous
import jax
import jax.numpy as jnp
from jax import lax
from jax.experimental import pallas as pl
from jax.experimental.pallas import tpu as pltpu


def _lrelu(x):
    return jnp.where(x >= 0, x, 0.01 * x)


def _s1a_body(x_ref, w1_ref, w2_ref, out_ref):
    x = x_ref[0]
    w1 = w1_ref[0]
    w2 = w2_ref[0]
    h1 = _lrelu(jnp.dot(w1, x, preferred_element_type=jnp.float32))
    h2 = _lrelu(jnp.dot(w2, h1, preferred_element_type=jnp.float32))
    out_ref[0] = h2


def _s1b_body(flat_ref, w3_ref, out_ref):
    out_ref[...] = jnp.dot(w3_ref[...], flat_ref[...],
                           preferred_element_type=jnp.float32)


def _argmax_rows(y):
    m = jnp.max(y, axis=0, keepdims=True)
    ri = lax.broadcasted_iota(jnp.int32, y.shape, 0)
    return jnp.min(jnp.where(y == m, ri, jnp.int32(2147483647)), axis=0)


def _onehot(e, n_experts, P):
    return (lax.broadcasted_iota(jnp.int32, (n_experts, P), 0)
            == e[None, :]).astype(jnp.float32)


def _wsel(tab, oh):
    return lax.dot_general(tab, oh, (((0,), (0,)), ((), ())),
                           preferred_element_type=jnp.float32)


def _matvec(wsel, v, ci, P):
    vb = jnp.reshape(jnp.broadcast_to(v[:, None, :], (ci, 32, P)),
                     (ci * 32, P))
    return jnp.sum((wsel * vb).reshape(ci, 32, P), axis=0)


def _s23_body(o_ref, x_ref, a1_ref, a2_ref, a3_ref,
              b1_ref, b2_ref, b3_ref, out_ref, i_s, acc_s):
    c = pl.program_id(1)
    xr = x_ref[0]
    ci, P = xr.shape

    @pl.when(c == 0)
    def _stage2():
        e1 = _argmax_rows(o_ref[...])
        oh = _onehot(e1, 16, P)
        y = _lrelu(_matvec(_wsel(a1_ref[0], oh), xr, ci, P))
        y = _lrelu(_matvec(_wsel(a2_ref[0], oh), y, 32, P))
        y = _matvec(_wsel(a3_ref[0], oh), y, 32, P)
        i2 = _argmax_rows(y)
        inds12 = e1 * 16 + (i2 - 8)
        i_s[0, :] = jnp.clip(inds12, 0, 255)
        i_s[1, :] = inds12
        acc_s[...] = jnp.zeros(acc_s.shape, jnp.float32)

    el = i_s[0, :] - c * 128
    ohc = _onehot(el, 128, P)
    t1 = jnp.swapaxes(b1_ref[...], 1, 2).reshape(128 * 32, ci)
    z1 = _lrelu(jnp.dot(t1, xr, preferred_element_type=jnp.float32))
    acc_s[...] += jnp.sum(z1.reshape(128, 32, P) * ohc[:, None, :], axis=0)

    @pl.when(c == 1)
    def _final():
        oh = _onehot(i_s[0, :], 256, P)
        y = _lrelu(_matvec(_wsel(b2_ref[0], oh), acc_s[...], 32, P))
        y = _matvec(_wsel(b3_ref[0], oh), y, 32, P)
        i3 = _argmax_rows(y)
        out_ref[0, 0, :] = jnp.clip(i_s[1, :] * 16 + (i3 - 8), 0, 4095)


def kernel(x_in, W1, b1, W2, b2, W3, b3, A1w, A1b, A2w, A2b, A3w, A3b,
           B1w, B1b, B2w, B2b, B3w, B3b):
    bs, ch_in, height, width = x_in.shape
    lat = W1.shape[1]
    c0 = 16

    xrows = jnp.transpose(x_in, (0, 2, 1, 3)).reshape(height, ch_in, width)
    h2 = pl.pallas_call(
        _s1a_body,
        grid=(height,),
        in_specs=[
            pl.BlockSpec((1, ch_in, width), lambda r: (r, 0, 0)),
            pl.BlockSpec((1, lat, ch_in), lambda r: (r, 0, 0)),
            pl.BlockSpec((1, lat, lat), lambda r: (r, 0, 0)),
        ],
        out_specs=pl.BlockSpec((1, lat, width), lambda r: (r, 0, 0)),
        out_shape=jax.ShapeDtypeStruct((height, lat, width), jnp.float32),
    )(xrows, W1, W2)

    flat = h2.reshape(height * lat, width)
    o = pl.pallas_call(
        _s1b_body,
        out_shape=jax.ShapeDtypeStruct((height * c0, width), jnp.float32),
    )(flat, W3)

    a1 = A1w.reshape(height, c0, ch_in * 32)
    a2 = A2w.reshape(height, c0, 32 * 32)
    a3 = A3w.reshape(height, c0, 32 * 32)
    b2r = B2w.reshape(height, 256, 32 * 32)
    b3r = B3w.reshape(height, 256, 32 * 32)

    out = pl.pallas_call(
        _s23_body,
        grid=(height, 2),
        in_specs=[
            pl.BlockSpec((c0, width), lambda r, c: (r, 0)),
            pl.BlockSpec((1, ch_in, width), lambda r, c: (r, 0, 0)),
            pl.BlockSpec((1, c0, ch_in * 32), lambda r, c: (r, 0, 0)),
            pl.BlockSpec((1, c0, 32 * 32), lambda r, c: (r, 0, 0)),
            pl.BlockSpec((1, c0, 32 * 32), lambda r, c: (r, 0, 0)),
            pl.BlockSpec((128, ch_in, 32), lambda r, c: (2 * r + c, 0, 0)),
            pl.BlockSpec((1, 256, 32 * 32), lambda r, c: (r, 0, 0)),
            pl.BlockSpec((1, 256, 32 * 32), lambda r, c: (r, 0, 0)),
        ],
        out_specs=pl.BlockSpec((1, 1, width), lambda r, c: (r, 0, 0)),
        out_shape=jax.ShapeDtypeStruct((height, 1, width), jnp.int32),
        scratch_shapes=[
            pltpu.VMEM((8, width), jnp.int32),
            pltpu.VMEM((lat, width), jnp.float32),
        ],
    )(o, xrows, a1, a2, a3, B1w, b2r, b3r)

    return out.reshape(bs, 1, height, width)


# speedup vs baseline: 1.6412x; 1.3077x over previous
import jax
import jax.numpy as jnp
from jax import lax
from jax.experimental import pallas as pl


def _lrelu(x):
    return jnp.where(x >= 0, x, 0.01 * x)


def _s1a_body(x_ref, w1_ref, w2_ref, out_ref):
    x = x_ref[0]
    w1 = w1_ref[0]
    w2 = w2_ref[0]
    h1 = _lrelu(jnp.dot(w1, x, preferred_element_type=jnp.float32))
    h2 = _lrelu(jnp.dot(w2, h1, preferred_element_type=jnp.float32))
    out_ref[0] = h2


def _s1b_body(flat_ref, w3_ref, out_ref):
    out_ref[...] = jnp.dot(w3_ref[...], flat_ref[...],
                           preferred_element_type=jnp.float32)


def _argmax_rows(y):
    m = jnp.max(y, axis=0, keepdims=True)
    ri = lax.broadcasted_iota(jnp.int32, y.shape, 0)
    return jnp.min(jnp.where(y == m, ri, jnp.int32(2147483647)), axis=0)


def _expert_mlp(xr, e, w1r, w2r, w3r, n_experts):
    ci, P = xr.shape
    oh = (lax.broadcasted_iota(jnp.int32, (n_experts, P), 0)
          == e[None, :]).astype(jnp.float32)
    w1s = lax.dot_general(w1r, oh, (((0,), (0,)), ((), ())),
                          preferred_element_type=jnp.float32)
    y = jnp.sum((w1s * jnp.reshape(
        jnp.broadcast_to(xr[:, None, :], (ci, 32, P)), (ci * 32, P))
    ).reshape(ci, 32, P), axis=0)
    y = _lrelu(y)
    w2s = lax.dot_general(w2r, oh, (((0,), (0,)), ((), ())),
                          preferred_element_type=jnp.float32)
    y = _lrelu(jnp.sum((w2s * jnp.reshape(
        jnp.broadcast_to(y[:, None, :], (32, 32, P)), (1024, P))
    ).reshape(32, 32, P), axis=0))
    w3s = lax.dot_general(w3r, oh, (((0,), (0,)), ((), ())),
                          preferred_element_type=jnp.float32)
    y3 = jnp.sum((w3s * jnp.reshape(
        jnp.broadcast_to(y[:, None, :], (32, 32, P)), (1024, P))
    ).reshape(32, 32, P), axis=0)
    return y3


def _s23_body(o_ref, x_ref, a1_ref, a2_ref, a3_ref,
              b1_ref, b2_ref, b3_ref, out_ref):
    xr = x_ref[0]
    orow = o_ref[...]
    e1 = _argmax_rows(orow)

    y = _expert_mlp(xr, e1, a1_ref[0], a2_ref[0], a3_ref[0], 16)
    i2 = _argmax_rows(y)
    inds12 = e1 * 16 + (i2 - 8)
    e12 = jnp.clip(inds12, 0, 255)

    y = _expert_mlp(xr, e12, b1_ref[0], b2_ref[0], b3_ref[0], 256)
    i3 = _argmax_rows(y)
    out_ref[0, 0, :] = jnp.clip(inds12 * 16 + (i3 - 8), 0, 4095)


def kernel(x_in, W1, b1, W2, b2, W3, b3, A1w, A1b, A2w, A2b, A3w, A3b,
           B1w, B1b, B2w, B2b, B3w, B3b):
    bs, ch_in, height, width = x_in.shape
    lat = W1.shape[1]
    c0 = 16

    xrows = jnp.transpose(x_in, (0, 2, 1, 3)).reshape(height, ch_in, width)
    h2 = pl.pallas_call(
        _s1a_body,
        grid=(height,),
        in_specs=[
            pl.BlockSpec((1, ch_in, width), lambda r: (r, 0, 0)),
            pl.BlockSpec((1, lat, ch_in), lambda r: (r, 0, 0)),
            pl.BlockSpec((1, lat, lat), lambda r: (r, 0, 0)),
        ],
        out_specs=pl.BlockSpec((1, lat, width), lambda r: (r, 0, 0)),
        out_shape=jax.ShapeDtypeStruct((height, lat, width), jnp.float32),
    )(xrows, W1, W2)

    flat = h2.reshape(height * lat, width)
    o = pl.pallas_call(
        _s1b_body,
        out_shape=jax.ShapeDtypeStruct((height * c0, width), jnp.float32),
    )(flat, W3)

    a1 = A1w.reshape(height, c0, ch_in * 32)
    a2 = A2w.reshape(height, c0, 32 * 32)
    a3 = A3w.reshape(height, c0, 32 * 32)
    b1r = B1w.reshape(height, 256, ch_in * 32)
    b2r = B2w.reshape(height, 256, 32 * 32)
    b3r = B3w.reshape(height, 256, 32 * 32)

    out = pl.pallas_call(
        _s23_body,
        grid=(height,),
        in_specs=[
            pl.BlockSpec((c0, width), lambda r: (r, 0)),
            pl.BlockSpec((1, ch_in, width), lambda r: (r, 0, 0)),
            pl.BlockSpec((1, c0, ch_in * 32), lambda r: (r, 0, 0)),
            pl.BlockSpec((1, c0, 32 * 32), lambda r: (r, 0, 0)),
            pl.BlockSpec((1, c0, 32 * 32), lambda r: (r, 0, 0)),
            pl.BlockSpec((1, 256, ch_in * 32), lambda r: (r, 0, 0)),
            pl.BlockSpec((1, 256, 32 * 32), lambda r: (r, 0, 0)),
            pl.BlockSpec((1, 256, 32 * 32), lambda r: (r, 0, 0)),
        ],
        out_specs=pl.BlockSpec((1, 1, width), lambda r: (r, 0, 0)),
        out_shape=jax.ShapeDtypeStruct((height, 1, width), jnp.int32),
    )(o, xrows, a1, a2, a3, b1r, b2r, b3r)

    return out.reshape(bs, 1, height, width)
